# Initial kernel scaffold; baseline (speedup 1.0000x reference)
import jax
import jax.numpy as jnp
from jax.experimental import pallas as pl


def kernel(x, edge_index, edge_attr, batch, W1, b1, g1, be1, rm1, rv1, W2, b2, g2, be2, rm2, rv2, W3, b3, g3, be3, rm3, rv3, Wm1, bm1, Wm2, bm2):
    raise NotImplementedError("write your pallas kernel here")


# speedup vs baseline: 9.1228x; 9.1228x over previous
import functools

import jax
import jax.numpy as jnp
from jax import lax
from jax.experimental import pallas as pl
from jax.experimental.pallas import tpu as pltpu
from jax.experimental.pallas import tpu_sc as plsc

N = 10000
E = 320000
NG = 64
NC = 2
NS = 16
NW = NC * NS
NPAD = 10240
DEGW = 128

_MESH = plsc.VectorSubcoreMesh(core_axis_name="c", subcore_axis_name="s")

_EW = E // NW
_C = 80
_NCH = _EW // _C


@functools.partial(
    pl.kernel,
    mesh=_MESH,
    out_type=jax.ShapeDtypeStruct((NC, NPAD, DEGW), jnp.float32),
    scratch_types=[
        pltpu.VMEM((_C,), jnp.int32),
        pltpu.VMEM((_C, DEGW), jnp.float32),
        pltpu.VMEM((128,), jnp.int32),
        pltpu.VMEM((128, DEGW), jnp.float32),
        pltpu.VMEM_SHARED((NPAD, DEGW), jnp.float32),
    ],
)
def _sc_deg(dst_hbm, degp_hbm, dstv, onesv, rowidx, zbuf, acc):
    cid = lax.axis_index("c")
    sid = lax.axis_index("s")
    wid = sid * NC + cid
    rpt = NPAD // NS
    lanes = lax.iota(jnp.int32, 16)

    @pl.loop(0, _C)
    def _ones(i):
        for u in range(DEGW // 16):
            onesv[i, pl.ds(u * 16, 16)] = jnp.ones((16,), jnp.float32)

    @pl.loop(0, 128)
    def _zeros(i):
        for u in range(DEGW // 16):
            zbuf[i, pl.ds(u * 16, 16)] = jnp.zeros((16,), jnp.float32)

    @pl.loop(0, rpt // 128)
    def _zero(j):
        base = sid * rpt + j * 128
        for q in range(8):
            rowidx[pl.ds(q * 16, 16)] = base + q * 16 + lanes
        pltpu.sync_copy(zbuf, acc.at[rowidx])

    plsc.subcore_barrier()

    ebase = wid * _EW

    @pl.loop(0, _NCH)
    def _edges(i):
        pltpu.sync_copy(dst_hbm.at[pl.ds(ebase + i * _C, _C)], dstv)
        pltpu.sync_copy(onesv, acc.at[dstv], add=True)

    plsc.subcore_barrier()

    @pl.loop(0, rpt // 128)
    def _wb(j):
        base = sid * rpt + j * 128
        for q in range(8):
            rowidx[pl.ds(q * 16, 16)] = base + q * 16 + lanes
        pltpu.sync_copy(acc.at[rowidx], zbuf)
        pltpu.sync_copy(zbuf, degp_hbm.at[cid, pl.ds(base, 128)])


def _make_agg(W):
    rpt = NPAD // NS
    zr = 128

    @functools.partial(
        pl.kernel,
        mesh=_MESH,
        out_type=jax.ShapeDtypeStruct((NC, NPAD, W), jnp.float32),
        scratch_types=[
            pltpu.VMEM((_C,), jnp.int32),
            pltpu.VMEM((_C,), jnp.int32),
            pltpu.VMEM((_C, W), jnp.float32),
            pltpu.VMEM((zr,), jnp.int32),
            pltpu.VMEM((zr, W), jnp.float32),
            pltpu.VMEM_SHARED((NPAD, W), jnp.float32),
            pltpu.SemaphoreType.DMA,
        ],
    )
    def agg(h_hbm, src_hbm, dst_hbm, out_hbm, srcv, dstv, rows, rowidx, zbuf,
            acc, sem):
        cid = lax.axis_index("c")
        sid = lax.axis_index("s")
        wid = sid * NC + cid
        lanes = lax.iota(jnp.int32, 16)

        @pl.loop(0, zr)
        def _zb(i):
            for u in range(W // 16):
                zbuf[i, pl.ds(u * 16, 16)] = jnp.zeros((16,), jnp.float32)

        @pl.loop(0, rpt // zr)
        def _zero(j):
            base = sid * rpt + j * zr
            for q in range(zr // 16):
                rowidx[pl.ds(q * 16, 16)] = base + q * 16 + lanes
            pltpu.sync_copy(zbuf, acc.at[rowidx])

        plsc.subcore_barrier()

        ebase = wid * _EW

        @pl.loop(0, _NCH)
        def _edges(i):
            pltpu.sync_copy(src_hbm.at[pl.ds(ebase + i * _C, _C)], srcv)
            pltpu.sync_copy(dst_hbm.at[pl.ds(ebase + i * _C, _C)], dstv)
            pltpu.async_copy(h_hbm.at[srcv], rows, sem).wait()
            pltpu.sync_copy(rows, acc.at[dstv], add=True)

        plsc.subcore_barrier()

        @pl.loop(0, rpt // zr)
        def _wb(j):
            base = sid * rpt + j * zr
            for q in range(zr // 16):
                rowidx[pl.ds(q * 16, 16)] = base + q * 16 + lanes
            pltpu.sync_copy(acc.at[rowidx], zbuf)
            pltpu.sync_copy(zbuf, out_hbm.at[cid, pl.ds(base, zr)])

    return agg


_sc_agg128 = _make_agg(128)


_NCHT = N // 16
_POOL_ITERS = -(-_NCHT // NW)


@functools.partial(
    pl.kernel,
    mesh=_MESH,
    compiler_params=pltpu.CompilerParams(needs_layout_passes=False),
    out_type=(
        jax.ShapeDtypeStruct((NW, NG, 64), jnp.float32),
        jax.ShapeDtypeStruct((NW, NG, 64), jnp.float32),
    ),
    scratch_types=[
        pltpu.VMEM((16,), jnp.int32),
        pltpu.VMEM((16, 64), jnp.float32),
        pltpu.VMEM((NG, 64), jnp.float32),
        pltpu.VMEM((NG, 64), jnp.float32),
    ],
)
def _sc_pool(h3_hbm, batch_hbm, psum_hbm, pmax_hbm, bv, hv, sacc, macc):
    cid = lax.axis_index("c")
    sid = lax.axis_index("s")
    wid = sid * NC + cid

    @pl.loop(0, NG)
    def _init(g):
        for u in range(4):
            sacc[g, pl.ds(u * 16, 16)] = jnp.zeros((16,), jnp.float32)
            macc[g, pl.ds(u * 16, 16)] = jnp.full((16,), -jnp.inf, jnp.float32)

    lanes = lax.iota(jnp.int32, 16)

    @pl.loop(0, _POOL_ITERS)
    def _chunk(ci):
        k = wid + ci * NW

        @pl.when(k < _NCHT)
        def _():
            pltpu.sync_copy(batch_hbm.at[pl.ds(k * 16, 16)], bv)
            pltpu.sync_copy(h3_hbm.at[pl.ds(k * 16, 16)], hv)
            for j in range(16):
                gj = plsc.load_gather(bv, [jnp.full((16,), j, jnp.int32)])
                for u in range(4):
                    col = u * 16 + lanes
                    row = hv[j, pl.ds(u * 16, 16)]
                    plsc.addupdate_scatter(sacc, [gj, col], row)
                    cur = plsc.load_gather(macc, [gj, col])
                    plsc.store_scatter(macc, [gj, col], jnp.maximum(cur, row))

    pltpu.sync_copy(sacc, psum_hbm.at[wid])
    pltpu.sync_copy(macc, pmax_hbm.at[wid])


_R = 2000


def _elu(v):
    return jnp.where(v > 0, v, jnp.exp(v) - 1.0)


def _dot(a, b):
    return jnp.dot(a.astype(jnp.bfloat16), b.astype(jnp.bfloat16),
                   preferred_element_type=jnp.float32)


def _bn(z, rm, rs, g, be):
    return ((z - rm) * rs) * g + be


def _tc1_body(degp, x, w1, dinv, hp1a, hp1b):
    dsum = degp[0, :, :1] + degp[1, :, :1] + 1.0
    di = lax.rsqrt(dsum)
    dinv[...] = di
    hp = di * _dot(x[...], w1[...])
    hp1a[...] = hp[:, :128]
    hp1b[...] = hp[:, 128:]


_tc1 = pl.pallas_call(
    _tc1_body,
    grid=(N // _R,),
    in_specs=[
        pl.BlockSpec((NC, _R, DEGW), lambda i: (0, i, 0)),
        pl.BlockSpec((_R, 128), lambda i: (i, 0)),
        pl.BlockSpec((128, 256), lambda i: (0, 0)),
    ],
    out_specs=[
        pl.BlockSpec((_R, 1), lambda i: (i, 0)),
        pl.BlockSpec((_R, 128), lambda i: (i, 0)),
        pl.BlockSpec((_R, 128), lambda i: (i, 0)),
    ],
    out_shape=[
        jax.ShapeDtypeStruct((N, 1), jnp.float32),
        jax.ShapeDtypeStruct((N, 128), jnp.float32),
        jax.ShapeDtypeStruct((N, 128), jnp.float32),
    ],
)


def _tc2_body(s1a, s1b, hp1a, hp1b, dinv, b1, rm1, rs1, g1, be1, w2, hp2):
    di = dinv[...]
    agga = di * (s1a[0] + s1a[1] + hp1a[...])
    aggb = di * (s1b[0] + s1b[1] + hp1b[...])
    z = jnp.concatenate([agga, aggb], axis=1) + b1[...]
    h1 = _elu(_bn(z, rm1[...], rs1[...], g1[...], be1[...]))
    hp2[...] = di * _dot(h1, w2[...])


_tc2 = pl.pallas_call(
    _tc2_body,
    grid=(N // _R,),
    in_specs=[
        pl.BlockSpec((NC, _R, 128), lambda i: (0, i, 0)),
        pl.BlockSpec((NC, _R, 128), lambda i: (0, i, 0)),
        pl.BlockSpec((_R, 128), lambda i: (i, 0)),
        pl.BlockSpec((_R, 128), lambda i: (i, 0)),
        pl.BlockSpec((_R, 1), lambda i: (i, 0)),
        pl.BlockSpec((1, 256), lambda i: (0, 0)),
        pl.BlockSpec((1, 256), lambda i: (0, 0)),
        pl.BlockSpec((1, 256), lambda i: (0, 0)),
        pl.BlockSpec((1, 256), lambda i: (0, 0)),
        pl.BlockSpec((1, 256), lambda i: (0, 0)),
        pl.BlockSpec((256, 128), lambda i: (0, 0)),
    ],
    out_specs=pl.BlockSpec((_R, 128), lambda i: (i, 0)),
    out_shape=jax.ShapeDtypeStruct((N, 128), jnp.float32),
)


def _tc3_body(s2, hp2, dinv, b2, rm2, rs2, g2, be2, w3p, hp3):
    di = dinv[...]
    z = di * (s2[0] + s2[1] + hp2[...]) + b2[...]
    h2 = _elu(_bn(z, rm2[...], rs2[...], g2[...], be2[...]))
    hp3[...] = di * _dot(h2, w3p[...])


_tc3 = pl.pallas_call(
    _tc3_body,
    grid=(N // _R,),
    in_specs=[
        pl.BlockSpec((NC, _R, 128), lambda i: (0, i, 0)),
        pl.BlockSpec((_R, 128), lambda i: (i, 0)),
        pl.BlockSpec((_R, 1), lambda i: (i, 0)),
        pl.BlockSpec((1, 128), lambda i: (0, 0)),
        pl.BlockSpec((1, 128), lambda i: (0, 0)),
        pl.BlockSpec((1, 128), lambda i: (0, 0)),
        pl.BlockSpec((1, 128), lambda i: (0, 0)),
        pl.BlockSpec((1, 128), lambda i: (0, 0)),
        pl.BlockSpec((128, 128), lambda i: (0, 0)),
    ],
    out_specs=pl.BlockSpec((_R, 128), lambda i: (i, 0)),
    out_shape=jax.ShapeDtypeStruct((N, 128), jnp.float32),
)


def _tc4_body(s3, hp3, dinv, b3, rm3, rs3, g3, be3, h3):
    di = dinv[...]
    agg = di * (s3[0] + s3[1] + hp3[...])
    z = agg[:, :64] + b3[...]
    h3[...] = _elu(_bn(z, rm3[...], rs3[...], g3[...], be3[...]))


_tc4 = pl.pallas_call(
    _tc4_body,
    grid=(N // _R,),
    in_specs=[
        pl.BlockSpec((NC, _R, 128), lambda i: (0, i, 0)),
        pl.BlockSpec((_R, 128), lambda i: (i, 0)),
        pl.BlockSpec((_R, 1), lambda i: (i, 0)),
        pl.BlockSpec((1, 64), lambda i: (0, 0)),
        pl.BlockSpec((1, 64), lambda i: (0, 0)),
        pl.BlockSpec((1, 64), lambda i: (0, 0)),
        pl.BlockSpec((1, 64), lambda i: (0, 0)),
        pl.BlockSpec((1, 64), lambda i: (0, 0)),
    ],
    out_specs=pl.BlockSpec((_R, 64), lambda i: (i, 0)),
    out_shape=jax.ShapeDtypeStruct((N, 64), jnp.float32),
)


def _tcf_body(psum, pmax, batchr, wm1, bm1, wm2, bm2, out):
    b = batchr[...]
    gids = lax.broadcasted_iota(jnp.int32, (NG, N), 0)
    mask = (b == gids).astype(jnp.float32)
    cnt = jnp.sum(mask, axis=1, keepdims=True)
    mean = jnp.sum(psum[...], axis=0) / jnp.maximum(cnt, 1.0)
    mx = jnp.max(pmax[...], axis=0)
    g = jnp.concatenate([mean, mx], axis=1)
    hid = _elu(_dot(g, wm1[...]) + bm1[...])
    out[...] = _dot(hid, wm2[...]) + bm2[...]


_tcf = pl.pallas_call(
    _tcf_body,
    out_shape=jax.ShapeDtypeStruct((NG, 2), jnp.float32),
)


def kernel(x, edge_index, edge_attr, batch,
           W1, b1, g1, be1, rm1, rv1,
           W2, b2, g2, be2, rm2, rv2,
           W3, b3, g3, be3, rm3, rv3,
           Wm1, bm1, Wm2, bm2):
    src = edge_index[0]
    dst = edge_index[1]

    r2 = lambda v: v.reshape(1, -1)
    rs1 = r2(lax.rsqrt(rv1 + 1e-5))
    rs2 = r2(lax.rsqrt(rv2 + 1e-5))
    rs3 = r2(lax.rsqrt(rv3 + 1e-5))
    W3p = jnp.concatenate([W3, jnp.zeros((128, 64), jnp.float32)], axis=1)

    degp = _sc_deg(dst)
    dinv, hp1a, hp1b = _tc1(degp, x, W1)
    s1a = _sc_agg128(hp1a, src, dst)
    s1b = _sc_agg128(hp1b, src, dst)
    hp2 = _tc2(s1a, s1b, hp1a, hp1b, dinv,
               r2(b1), r2(rm1), rs1, r2(g1), r2(be1), W2)
    s2 = _sc_agg128(hp2, src, dst)
    hp3 = _tc3(s2, hp2, dinv, r2(b2), r2(rm2), rs2, r2(g2), r2(be2), W3p)
    s3 = _sc_agg128(hp3, src, dst)
    h3 = _tc4(s3, hp3, dinv, r2(b3), r2(rm3), rs3, r2(g3), r2(be3))
    psum, pmax = _sc_pool(h3, batch)
    out = _tcf(psum, pmax, batch.reshape(1, -1),
               Wm1, bm1.reshape(1, -1), Wm2, bm2.reshape(1, -1))
    return out
